# baseline (speedup 1.0000x reference)
import functools

import jax
import jax.numpy as jnp
from jax import lax
from jax.experimental import pallas as pl
from jax.experimental.pallas import tpu as pltpu
from jax.experimental.pallas import tpu_sc as plsc

NPTS = 10000
NP = 10240
K_NB = 16
RADIUS = 1.5
KS = 4
NCHUNK = NP // 128
INTMAX = 2**31 - 1
D2CLIP = 2.26
KEYSCALE = float((2**18 - 1) / D2CLIP)
NW = 32
QPW = NP // NW
_SC_PARAMS = pltpu.CompilerParams(
    needs_layout_passes=False, use_tc_tiling_on_sc=False)


NLVL = 4
NSETS = 6


def _knn_body(posl_ref, posq_ref, out_ref):
    i = pl.program_id(0)
    qx = posq_ref[:, 0:1]
    qy = posq_ref[:, 1:2]
    qz = posq_ref[:, 2:3]
    qidx = i * 8 + lax.broadcasted_iota(jnp.int32, (8, 1), 0)

    lane = lax.broadcasted_iota(jnp.int32, (8, 128), 1)
    pools = [[jnp.full((8, 128), INTMAX, jnp.int32) for _ in range(NLVL)]
             for _ in range(NSETS)]
    for v in range(NCHUNK):
        px = posl_ref[v]
        py = posl_ref[NCHUNK + v]
        pz = posl_ref[2 * NCHUNK + v]
        dx = qx - px
        dy = qy - py
        dz = qz - pz
        d2 = dx * dx + dy * dy + dz * dz
        j = v * 128 + lane
        q18 = (jnp.minimum(d2, D2CLIP) * KEYSCALE).astype(jnp.int32)
        key = jnp.left_shift(q18 - 2**17, 14) | j
        t = jnp.where(j == qidx, INTMAX, key)
        pool = pools[v % NSETS]
        for l in range(NLVL):
            lo = jnp.minimum(pool[l], t)
            t = jnp.maximum(pool[l], t)
            pool[l] = lo

    idxs = []
    for _ in range(K_NB):
        m0 = pools[0][0]
        for s in range(1, NSETS):
            m0 = jnp.minimum(m0, pools[s][0])
        m = jnp.min(m0, axis=1, keepdims=True)
        idxs.append(jnp.bitwise_and(m, 16383))
        for s in range(NSETS):
            pool = pools[s]
            c = pool[0] == m
            for l in range(NLVL - 1):
                pool[l] = jnp.where(c, pool[l + 1], pool[l])
            pool[NLVL - 1] = jnp.where(c, INTMAX, pool[NLVL - 1])
    out_ref[...] = jnp.concatenate(idxs, axis=1)


def _knn(posl, posq):
    return pl.pallas_call(
        _knn_body,
        grid=(NP // 8,),
        in_specs=[
            pl.BlockSpec((3 * NCHUNK, 8, 128), lambda i: (0, 0, 0)),
            pl.BlockSpec((8, 3), lambda i: (i, 0)),
        ],
        out_specs=pl.BlockSpec((8, K_NB), lambda i: (i, 0)),
        out_shape=jax.ShapeDtypeStruct((NP, K_NB), jnp.int32),
    )(posl, posq)


def _sqrt_sc(a):
    b = lax.bitcast_convert_type(a, jnp.int32)
    y = lax.bitcast_convert_type(
        jnp.right_shift(b, 1) + jnp.int32(0x1FBD1DF5), jnp.float32)
    for _ in range(3):
        y = 0.5 * (y + a / y)
    return y


def _geom_body(posx_h, posy_h, posz_h, nbr_h, s8_h, gidx_h,
               px_v, py_v, pz_v, nbr_v, s8_v, gi_v):
    wid = lax.axis_index("s") * 2 + lax.axis_index("c")
    base = wid * QPW
    pltpu.sync_copy(posx_h, px_v)
    pltpu.sync_copy(posy_h, py_v)
    pltpu.sync_copy(posz_h, pz_v)
    pltpu.sync_copy(nbr_h.at[pl.ds(base * K_NB, QPW * K_NB)], nbr_v)

    cq = 16

    def chunk(ci, _):
        def one_q(iq, _):
            q = ci * cq + iq
            qsplat = jnp.full((16,), base + q, jnp.int32)
            qx = plsc.load_gather(px_v, [qsplat])
            qy = plsc.load_gather(py_v, [qsplat])
            qz = plsc.load_gather(pz_v, [qsplat])
            jdx = nbr_v[pl.ds(q * K_NB, K_NB)]
            pjx = plsc.load_gather(px_v, [jdx])
            pjy = plsc.load_gather(py_v, [jdx])
            pjz = plsc.load_gather(pz_v, [jdx])
            rx = (pjx - qx) * (1.0 / RADIUS)
            ry = (pjy - qy) * (1.0 / RADIUS)
            rz = (pjz - qz) * (1.0 / RADIUS)
            r2 = rx * rx + ry * ry + rz * rz
            r2c = jnp.minimum(jnp.maximum(r2, 0.0), 1.0)
            om = 1.0 - r2c
            win = om * om * om
            win = jnp.where(r2 <= 1.0, win, 0.0)
            norm2 = _sqrt_sc(jnp.maximum(r2, 1e-12))
            ninf = jnp.maximum(
                jnp.maximum(jnp.abs(rx), jnp.abs(ry)),
                jnp.maximum(jnp.abs(rz), 1e-12))
            s = norm2 / ninf
            ux = jnp.clip((jnp.clip(rx * s, -1.0, 1.0) + 1.0) * 1.5, 0.0, 3.0)
            uy = jnp.clip((jnp.clip(ry * s, -1.0, 1.0) + 1.0) * 1.5, 0.0, 3.0)
            uz = jnp.clip((jnp.clip(rz * s, -1.0, 1.0) + 1.0) * 1.5, 0.0, 3.0)
            ix = jnp.minimum(ux.astype(jnp.int32), 2)
            iy = jnp.minimum(uy.astype(jnp.int32), 2)
            iz = jnp.minimum(uz.astype(jnp.int32), 2)
            fx = ux - ix.astype(jnp.float32)
            fy = uy - iy.astype(jnp.float32)
            fz = uz - iz.astype(jnp.float32)
            j64 = jdx * 64
            gb = ix * 16 + iy * 4 + iz
            for c in range(8):
                cx, cy, cz = (c >> 2) & 1, (c >> 1) & 1, c & 1
                wx = fx if cx else 1.0 - fx
                wy = fy if cy else 1.0 - fy
                wz = fz if cz else 1.0 - fz
                w = win * wx * wy * wz
                g = gb + (cx * 16 + cy * 4 + cz)
                off = iq * 128 + c * 16
                s8_v[pl.ds(off, 16)] = w
                gi_v[pl.ds(off, 16)] = j64 + g
            return 0

        lax.fori_loop(0, cq, one_q, 0)
        hb = (base + ci * cq) * 128
        pltpu.sync_copy(s8_v, s8_h.at[pl.ds(hb, cq * 128)])
        pltpu.sync_copy(gi_v, gidx_h.at[pl.ds(hb, cq * 128)])
        return 0

    lax.fori_loop(0, QPW // cq, chunk, 0)


def _geometry(posx, posy, posz, nbr):
    mesh = plsc.VectorSubcoreMesh(core_axis_name="c", subcore_axis_name="s")
    f = pl.kernel(
        _geom_body,
        out_type=(
            jax.ShapeDtypeStruct((NP * 128,), jnp.float32),
            jax.ShapeDtypeStruct((NP * 128,), jnp.int32),
        ),
        mesh=mesh,
        compiler_params=_SC_PARAMS,
        scratch_types=[
            pltpu.VMEM((NP,), jnp.float32),
            pltpu.VMEM((NP,), jnp.float32),
            pltpu.VMEM((NP,), jnp.float32),
            pltpu.VMEM((QPW * K_NB,), jnp.int32),
            pltpu.VMEM((16 * 128,), jnp.float32),
            pltpu.VMEM((16 * 128,), jnp.int32),
        ],
    )
    return f(posx, posy, posz, nbr.reshape(NP * K_NB))


def _make_fma_body(co):
    nacc = co // 16
    cqs = 8

    def body(p_h, gidx_h, s8_h, out_h, idx_v, s_v, rows_v, acc_v, gsem):
        wid = lax.axis_index("s") * 2 + lax.axis_index("c")
        base = wid * QPW

        def chunk(ci, _):
            q0 = base + ci * cqs
            pltpu.sync_copy(gidx_h.at[pl.ds(q0 * 128, cqs * 128)], idx_v)
            pltpu.sync_copy(s8_h.at[pl.ds(q0 * 128, cqs * 128)], s_v)
            cps = [
                pltpu.async_copy(
                    p_h.at[idx_v.at[pl.ds(j * 128, 128)]],
                    rows_v.at[pl.ds(j * 128, 128)], gsem)
                for j in range(cqs)
            ]
            for cp in cps:
                cp.wait()

            lane16 = lax.broadcasted_iota(jnp.int32, (16,), 0)

            def one_q(j, _):
                def grp(g, accs):
                    sbase = j * 128 + g * 16
                    for t in range(16):
                        sb = plsc.load_gather(
                            s_v, [jnp.full((16,), sbase + t, jnp.int32)])
                        row = jnp.full((16,), sbase + t, jnp.int32)
                        accs = tuple(
                            accs[u] + sb * plsc.load_gather(
                                rows_v, [row, u * 16 + lane16])
                            for u in range(nacc))
                    return accs

                accs = lax.fori_loop(
                    0, 8, grp,
                    tuple(jnp.zeros((16,), jnp.float32) for _ in range(nacc)))
                for u in range(nacc):
                    acc_v[pl.ds(j * co + u * 16, 16)] = accs[u]
                return 0

            lax.fori_loop(0, cqs, one_q, 0)
            pltpu.sync_copy(
                acc_v, out_h.at[pl.ds(q0 * co, cqs * co)])
            return 0

        lax.fori_loop(0, QPW // cqs, chunk, 0)

    return body, cqs


def _sc_fma(p_rows, gidx, s8, co):
    body, cqs = _make_fma_body(co)
    mesh = plsc.VectorSubcoreMesh(core_axis_name="c", subcore_axis_name="s")
    f = pl.kernel(
        body,
        out_type=jax.ShapeDtypeStruct((NP * co,), jnp.float32),
        mesh=mesh,
        compiler_params=_SC_PARAMS,
        scratch_types=[
            pltpu.VMEM((cqs * 128,), jnp.int32),
            pltpu.VMEM((cqs * 128,), jnp.float32),
            pltpu.VMEM((cqs * 128, co), jnp.float32),
            pltpu.VMEM((cqs * co,), jnp.float32),
            pltpu.SemaphoreType.DMA,
        ],
    )
    return f(p_rows, gidx, s8).reshape(NP, co)


def _mm_body(x_ref, w_ref, b_ref, o1_ref, o2_ref, *, n1):
    res = jnp.dot(x_ref[...], w_ref[...], preferred_element_type=jnp.float32)
    o1_ref[...] = res[:, :n1]
    o2_ref[...] = res[:, n1:] + b_ref[...]


def _mm(x, w, bias2, n1, n2, bm=256):
    m, k = x.shape
    return pl.pallas_call(
        functools.partial(_mm_body, n1=n1),
        grid=(m // bm,),
        in_specs=[
            pl.BlockSpec((bm, k), lambda i: (i, 0)),
            pl.BlockSpec((k, n1 + n2), lambda i: (0, 0)),
            pl.BlockSpec((1, n2), lambda i: (0, 0)),
        ],
        out_specs=[
            pl.BlockSpec((bm, n1), lambda i: (i, 0)),
            pl.BlockSpec((bm, n2), lambda i: (i, 0)),
        ],
        out_shape=[
            jax.ShapeDtypeStruct((m, n1), jnp.float32),
            jax.ShapeDtypeStruct((m, n2), jnp.float32),
        ],
    )(x, w, bias2)


def _combine_body(c_ref, d_ref, r_ref, o_ref):
    o_ref[...] = jnp.maximum(c_ref[...] + d_ref[...] + r_ref[...], 0.0)


def _combine_body2(c_ref, d_ref, o_ref):
    o_ref[...] = jnp.maximum(c_ref[...] + d_ref[...], 0.0)


def _combine(c, d, res=None, bm=512):
    m, n = c.shape
    if res is None:
        return pl.pallas_call(
            _combine_body2,
            grid=(m // bm,),
            in_specs=[pl.BlockSpec((bm, n), lambda i: (i, 0))] * 2,
            out_specs=pl.BlockSpec((bm, n), lambda i: (i, 0)),
            out_shape=jax.ShapeDtypeStruct((m, n), jnp.float32),
        )(c, d)
    return pl.pallas_call(
        _combine_body,
        grid=(m // bm,),
        in_specs=[pl.BlockSpec((bm, n), lambda i: (i, 0))] * 3,
        out_specs=pl.BlockSpec((bm, n), lambda i: (i, 0)),
        out_shape=jax.ShapeDtypeStruct((m, n), jnp.float32),
    )(c, d, res)


def _prep_w(Wc, Wd, bd, cin_pad, co_pad):
    ks3, cin, co = Wc.shape
    wc = jnp.zeros((ks3, cin_pad, co_pad), jnp.float32).at[:, :cin, :co].set(Wc)
    wcf = wc.transpose(1, 0, 2).reshape(cin_pad, ks3 * co_pad)
    wd = jnp.zeros((cin_pad, co_pad), jnp.float32).at[:cin, :co].set(Wd)
    w = jnp.concatenate([wcf, wd], axis=1)
    bias = jnp.zeros((1, co_pad), jnp.float32).at[0, :co].set(bd)
    return w, bias


def kernel(pos, vel, mass, extra_feats,
           Wc0, Wd0, bd0, Wc1, Wd1, bd1, Wc2, Wd2, bd2, Wc3, Wd3, bd3):
    posp = jnp.full((NP, 3), 1e4, jnp.float32).at[:NPTS].set(pos)
    posl = posp.T.reshape(3, NCHUNK, 1, 128).reshape(3 * NCHUNK, 1, 128)
    posl = jnp.broadcast_to(posl, (3 * NCHUNK, 8, 128))

    nbr = _knn(posl, posp)

    posx, posy, posz = posp[:, 0], posp[:, 1], posp[:, 2]
    s8, gidx = _geometry(posx, posy, posz, nbr)

    feats = jnp.concatenate([mass[:, None], vel, extra_feats], axis=-1)
    x = jnp.zeros((NP, 16), jnp.float32).at[:NPTS, :13].set(feats)

    def layer(x, Wc, Wd, bd, cin_pad, co_pad):
        w, bias = _prep_w(Wc, Wd, bd, cin_pad, co_pad)
        p, d = _mm(x, w, bias, 64 * co_pad, co_pad)
        p_rows = p.reshape(NP * 64, co_pad)
        c = _sc_fma(p_rows, gidx, s8, co_pad)
        return c, d

    c0, d0 = layer(x, Wc0, Wd0, bd0, 16, 32)
    x1 = jnp.concatenate([c0, d0], axis=1)
    c1, d1 = layer(x1, Wc1, Wd1, bd1, 64, 64)
    x2 = _combine(c1, d1, x1)
    c2, d2 = layer(x2, Wc2, Wd2, bd2, 64, 64)
    x3 = _combine(c2, d2, x2)
    c3, d3 = layer(x3, Wc3, Wd3, bd3, 64, 16)
    out = _combine(c3, d3)
    return out[:NPTS, :3]


# speedup vs baseline: 1.0093x; 1.0093x over previous
import functools

import jax
import jax.numpy as jnp
from jax import lax
from jax.experimental import pallas as pl
from jax.experimental.pallas import tpu as pltpu
from jax.experimental.pallas import tpu_sc as plsc

NPTS = 10000
NP = 10240
K_NB = 16
RADIUS = 1.5
KS = 4
NCHUNK = NP // 128
INTMAX = 2**31 - 1
D2CLIP = 2.26
KEYSCALE = float((2**18 - 1) / D2CLIP)
NW = 32
QPW = NP // NW
_SC_PARAMS = pltpu.CompilerParams(
    needs_layout_passes=False, use_tc_tiling_on_sc=False)


NLVL = 4
NSETS = 3


def _knn_body(posl_ref, posq_ref, out_ref):
    i = pl.program_id(0)
    qx = posq_ref[:, 0:1]
    qy = posq_ref[:, 1:2]
    qz = posq_ref[:, 2:3]
    qidx = i * 8 + lax.broadcasted_iota(jnp.int32, (8, 1), 0)

    lane = lax.broadcasted_iota(jnp.int32, (8, 128), 1)
    pools = [[jnp.full((8, 128), INTMAX, jnp.int32) for _ in range(NLVL)]
             for _ in range(NSETS)]
    for v in range(NCHUNK):
        px = posl_ref[v]
        py = posl_ref[NCHUNK + v]
        pz = posl_ref[2 * NCHUNK + v]
        dx = qx - px
        dy = qy - py
        dz = qz - pz
        d2 = dx * dx + dy * dy + dz * dz
        j = v * 128 + lane
        q18 = (jnp.minimum(d2, D2CLIP) * KEYSCALE).astype(jnp.int32)
        key = jnp.left_shift(q18 - 2**17, 14) | j
        t = jnp.where(j == qidx, INTMAX, key)
        pool = pools[v % NSETS]
        for l in range(NLVL):
            lo = jnp.minimum(pool[l], t)
            t = jnp.maximum(pool[l], t)
            pool[l] = lo

    idxs = []
    for _ in range(K_NB):
        m0 = pools[0][0]
        for s in range(1, NSETS):
            m0 = jnp.minimum(m0, pools[s][0])
        m = jnp.min(m0, axis=1, keepdims=True)
        idxs.append(jnp.bitwise_and(m, 16383))
        for s in range(NSETS):
            pool = pools[s]
            c = pool[0] == m
            for l in range(NLVL - 1):
                pool[l] = jnp.where(c, pool[l + 1], pool[l])
            pool[NLVL - 1] = jnp.where(c, INTMAX, pool[NLVL - 1])
    out_ref[...] = jnp.concatenate(idxs, axis=1)


def _knn(posl, posq):
    return pl.pallas_call(
        _knn_body,
        grid=(NP // 8,),
        in_specs=[
            pl.BlockSpec((3 * NCHUNK, 8, 128), lambda i: (0, 0, 0)),
            pl.BlockSpec((8, 3), lambda i: (i, 0)),
        ],
        out_specs=pl.BlockSpec((8, K_NB), lambda i: (i, 0)),
        out_shape=jax.ShapeDtypeStruct((NP, K_NB), jnp.int32),
    )(posl, posq)


def _sqrt_sc(a):
    b = lax.bitcast_convert_type(a, jnp.int32)
    y = lax.bitcast_convert_type(
        jnp.right_shift(b, 1) + jnp.int32(0x1FBD1DF5), jnp.float32)
    for _ in range(3):
        y = 0.5 * (y + a / y)
    return y


def _geom_body(posx_h, posy_h, posz_h, nbr_h, s8_h, gidx_h,
               px_v, py_v, pz_v, nbr_v, s8_v, gi_v):
    wid = lax.axis_index("s") * 2 + lax.axis_index("c")
    base = wid * QPW
    pltpu.sync_copy(posx_h, px_v)
    pltpu.sync_copy(posy_h, py_v)
    pltpu.sync_copy(posz_h, pz_v)
    pltpu.sync_copy(nbr_h.at[pl.ds(base * K_NB, QPW * K_NB)], nbr_v)

    cq = 16

    def chunk(ci, _):
        def one_q(iq, _):
            q = ci * cq + iq
            qsplat = jnp.full((16,), base + q, jnp.int32)
            qx = plsc.load_gather(px_v, [qsplat])
            qy = plsc.load_gather(py_v, [qsplat])
            qz = plsc.load_gather(pz_v, [qsplat])
            jdx = nbr_v[pl.ds(q * K_NB, K_NB)]
            pjx = plsc.load_gather(px_v, [jdx])
            pjy = plsc.load_gather(py_v, [jdx])
            pjz = plsc.load_gather(pz_v, [jdx])
            rx = (pjx - qx) * (1.0 / RADIUS)
            ry = (pjy - qy) * (1.0 / RADIUS)
            rz = (pjz - qz) * (1.0 / RADIUS)
            r2 = rx * rx + ry * ry + rz * rz
            r2c = jnp.minimum(jnp.maximum(r2, 0.0), 1.0)
            om = 1.0 - r2c
            win = om * om * om
            win = jnp.where(r2 <= 1.0, win, 0.0)
            norm2 = _sqrt_sc(jnp.maximum(r2, 1e-12))
            ninf = jnp.maximum(
                jnp.maximum(jnp.abs(rx), jnp.abs(ry)),
                jnp.maximum(jnp.abs(rz), 1e-12))
            s = norm2 / ninf
            ux = jnp.clip((jnp.clip(rx * s, -1.0, 1.0) + 1.0) * 1.5, 0.0, 3.0)
            uy = jnp.clip((jnp.clip(ry * s, -1.0, 1.0) + 1.0) * 1.5, 0.0, 3.0)
            uz = jnp.clip((jnp.clip(rz * s, -1.0, 1.0) + 1.0) * 1.5, 0.0, 3.0)
            ix = jnp.minimum(ux.astype(jnp.int32), 2)
            iy = jnp.minimum(uy.astype(jnp.int32), 2)
            iz = jnp.minimum(uz.astype(jnp.int32), 2)
            fx = ux - ix.astype(jnp.float32)
            fy = uy - iy.astype(jnp.float32)
            fz = uz - iz.astype(jnp.float32)
            j64 = jdx * 64
            gb = ix * 16 + iy * 4 + iz
            for c in range(8):
                cx, cy, cz = (c >> 2) & 1, (c >> 1) & 1, c & 1
                wx = fx if cx else 1.0 - fx
                wy = fy if cy else 1.0 - fy
                wz = fz if cz else 1.0 - fz
                w = win * wx * wy * wz
                g = gb + (cx * 16 + cy * 4 + cz)
                off = iq * 128 + c * 16
                s8_v[pl.ds(off, 16)] = w
                gi_v[pl.ds(off, 16)] = j64 + g
            return 0

        lax.fori_loop(0, cq, one_q, 0)
        hb = (base + ci * cq) * 128
        pltpu.sync_copy(s8_v, s8_h.at[pl.ds(hb, cq * 128)])
        pltpu.sync_copy(gi_v, gidx_h.at[pl.ds(hb, cq * 128)])
        return 0

    lax.fori_loop(0, QPW // cq, chunk, 0)


def _geometry(posx, posy, posz, nbr):
    mesh = plsc.VectorSubcoreMesh(core_axis_name="c", subcore_axis_name="s")
    f = pl.kernel(
        _geom_body,
        out_type=(
            jax.ShapeDtypeStruct((NP * 128,), jnp.float32),
            jax.ShapeDtypeStruct((NP * 128,), jnp.int32),
        ),
        mesh=mesh,
        compiler_params=_SC_PARAMS,
        scratch_types=[
            pltpu.VMEM((NP,), jnp.float32),
            pltpu.VMEM((NP,), jnp.float32),
            pltpu.VMEM((NP,), jnp.float32),
            pltpu.VMEM((QPW * K_NB,), jnp.int32),
            pltpu.VMEM((16 * 128,), jnp.float32),
            pltpu.VMEM((16 * 128,), jnp.int32),
        ],
    )
    return f(posx, posy, posz, nbr.reshape(NP * K_NB))


def _make_fma_body(co):
    nacc = co // 16
    cqs = 8

    def body(p_h, gidx_h, s8_h, out_h, idx_v, s_v, rows_v, acc_v, gsem):
        wid = lax.axis_index("s") * 2 + lax.axis_index("c")
        base = wid * QPW

        def chunk(ci, _):
            q0 = base + ci * cqs
            pltpu.sync_copy(gidx_h.at[pl.ds(q0 * 128, cqs * 128)], idx_v)
            pltpu.sync_copy(s8_h.at[pl.ds(q0 * 128, cqs * 128)], s_v)
            cps = [
                pltpu.async_copy(
                    p_h.at[idx_v.at[pl.ds(j * 128, 128)]],
                    rows_v.at[pl.ds(j * 128, 128)], gsem)
                for j in range(cqs)
            ]
            for cp in cps:
                cp.wait()

            lane16 = lax.broadcasted_iota(jnp.int32, (16,), 0)

            def one_q(j, _):
                def grp(g, accs):
                    sbase = j * 128 + g * 16
                    for t in range(16):
                        sb = plsc.load_gather(
                            s_v, [jnp.full((16,), sbase + t, jnp.int32)])
                        row = jnp.full((16,), sbase + t, jnp.int32)
                        accs = tuple(
                            accs[u] + sb * plsc.load_gather(
                                rows_v, [row, u * 16 + lane16])
                            for u in range(nacc))
                    return accs

                accs = lax.fori_loop(
                    0, 8, grp,
                    tuple(jnp.zeros((16,), jnp.float32) for _ in range(nacc)))
                for u in range(nacc):
                    acc_v[pl.ds(j * co + u * 16, 16)] = accs[u]
                return 0

            lax.fori_loop(0, cqs, one_q, 0)
            pltpu.sync_copy(
                acc_v, out_h.at[pl.ds(q0 * co, cqs * co)])
            return 0

        lax.fori_loop(0, QPW // cqs, chunk, 0)

    return body, cqs


def _sc_fma(p_rows, gidx, s8, co):
    body, cqs = _make_fma_body(co)
    mesh = plsc.VectorSubcoreMesh(core_axis_name="c", subcore_axis_name="s")
    f = pl.kernel(
        body,
        out_type=jax.ShapeDtypeStruct((NP * co,), jnp.float32),
        mesh=mesh,
        compiler_params=_SC_PARAMS,
        scratch_types=[
            pltpu.VMEM((cqs * 128,), jnp.int32),
            pltpu.VMEM((cqs * 128,), jnp.float32),
            pltpu.VMEM((cqs * 128, co), jnp.float32),
            pltpu.VMEM((cqs * co,), jnp.float32),
            pltpu.SemaphoreType.DMA,
        ],
    )
    return f(p_rows, gidx, s8).reshape(NP, co)


def _mm_body(x_ref, w_ref, b_ref, o1_ref, o2_ref, *, n1):
    res = jnp.dot(x_ref[...], w_ref[...], preferred_element_type=jnp.float32)
    o1_ref[...] = res[:, :n1]
    o2_ref[...] = res[:, n1:] + b_ref[...]


def _mm(x, w, bias2, n1, n2, bm=256):
    m, k = x.shape
    return pl.pallas_call(
        functools.partial(_mm_body, n1=n1),
        grid=(m // bm,),
        in_specs=[
            pl.BlockSpec((bm, k), lambda i: (i, 0)),
            pl.BlockSpec((k, n1 + n2), lambda i: (0, 0)),
            pl.BlockSpec((1, n2), lambda i: (0, 0)),
        ],
        out_specs=[
            pl.BlockSpec((bm, n1), lambda i: (i, 0)),
            pl.BlockSpec((bm, n2), lambda i: (i, 0)),
        ],
        out_shape=[
            jax.ShapeDtypeStruct((m, n1), jnp.float32),
            jax.ShapeDtypeStruct((m, n2), jnp.float32),
        ],
    )(x, w, bias2)


def _combine_body(c_ref, d_ref, r_ref, o_ref):
    o_ref[...] = jnp.maximum(c_ref[...] + d_ref[...] + r_ref[...], 0.0)


def _combine_body2(c_ref, d_ref, o_ref):
    o_ref[...] = jnp.maximum(c_ref[...] + d_ref[...], 0.0)


def _combine(c, d, res=None, bm=512):
    m, n = c.shape
    if res is None:
        return pl.pallas_call(
            _combine_body2,
            grid=(m // bm,),
            in_specs=[pl.BlockSpec((bm, n), lambda i: (i, 0))] * 2,
            out_specs=pl.BlockSpec((bm, n), lambda i: (i, 0)),
            out_shape=jax.ShapeDtypeStruct((m, n), jnp.float32),
        )(c, d)
    return pl.pallas_call(
        _combine_body,
        grid=(m // bm,),
        in_specs=[pl.BlockSpec((bm, n), lambda i: (i, 0))] * 3,
        out_specs=pl.BlockSpec((bm, n), lambda i: (i, 0)),
        out_shape=jax.ShapeDtypeStruct((m, n), jnp.float32),
    )(c, d, res)


def _prep_w(Wc, Wd, bd, cin_pad, co_pad):
    ks3, cin, co = Wc.shape
    wc = jnp.zeros((ks3, cin_pad, co_pad), jnp.float32).at[:, :cin, :co].set(Wc)
    wcf = wc.transpose(1, 0, 2).reshape(cin_pad, ks3 * co_pad)
    wd = jnp.zeros((cin_pad, co_pad), jnp.float32).at[:cin, :co].set(Wd)
    w = jnp.concatenate([wcf, wd], axis=1)
    bias = jnp.zeros((1, co_pad), jnp.float32).at[0, :co].set(bd)
    return w, bias


def kernel(pos, vel, mass, extra_feats,
           Wc0, Wd0, bd0, Wc1, Wd1, bd1, Wc2, Wd2, bd2, Wc3, Wd3, bd3):
    posp = jnp.full((NP, 3), 1e4, jnp.float32).at[:NPTS].set(pos)
    posl = posp.T.reshape(3, NCHUNK, 1, 128).reshape(3 * NCHUNK, 1, 128)
    posl = jnp.broadcast_to(posl, (3 * NCHUNK, 8, 128))

    nbr = _knn(posl, posp)

    posx, posy, posz = posp[:, 0], posp[:, 1], posp[:, 2]
    s8, gidx = _geometry(posx, posy, posz, nbr)

    feats = jnp.concatenate([mass[:, None], vel, extra_feats], axis=-1)
    x = jnp.zeros((NP, 16), jnp.float32).at[:NPTS, :13].set(feats)

    def layer(x, Wc, Wd, bd, cin_pad, co_pad):
        w, bias = _prep_w(Wc, Wd, bd, cin_pad, co_pad)
        p, d = _mm(x, w, bias, 64 * co_pad, co_pad)
        p_rows = p.reshape(NP * 64, co_pad)
        c = _sc_fma(p_rows, gidx, s8, co_pad)
        return c, d

    c0, d0 = layer(x, Wc0, Wd0, bd0, 16, 32)
    x1 = jnp.concatenate([c0, d0], axis=1)
    c1, d1 = layer(x1, Wc1, Wd1, bd1, 64, 64)
    x2 = _combine(c1, d1, x1)
    c2, d2 = layer(x2, Wc2, Wd2, bd2, 64, 64)
    x3 = _combine(c2, d2, x2)
    c3, d3 = layer(x3, Wc3, Wd3, bd3, 64, 16)
    out = _combine(c3, d3)
    return out[:NPTS, :3]


# speedup vs baseline: 1.8045x; 1.7879x over previous
import functools

import jax
import jax.numpy as jnp
from jax import lax
from jax.experimental import pallas as pl
from jax.experimental.pallas import tpu as pltpu
from jax.experimental.pallas import tpu_sc as plsc

NPTS = 10000
NP = 10240
K_NB = 16
RADIUS = 1.5
KS = 4
NCHUNK = NP // 128
INTMAX = 2**31 - 1
D2CLIP = 2.26
KEYSCALE = float((2**18 - 1) / D2CLIP)
NW = 32
QPW = NP // NW
_SC_PARAMS = pltpu.CompilerParams(
    needs_layout_passes=False, use_tc_tiling_on_sc=False)


NLVL = 4
NSETS = 3


QG = 4
MLVL = 6


def _knn_phase1(posl_ref, qx, qy, qz, qidx, lane):
    pools = [[jnp.full((8, 128), INTMAX, jnp.int32) for _ in range(NLVL)]
             for _ in range(NSETS)]
    for v in range(NCHUNK):
        px = posl_ref[v]
        py = posl_ref[NCHUNK + v]
        pz = posl_ref[2 * NCHUNK + v]
        dx = qx - px
        dy = qy - py
        dz = qz - pz
        d2 = dx * dx + dy * dy + dz * dz
        j = v * 128 + lane
        q18 = (jnp.minimum(d2, D2CLIP) * KEYSCALE).astype(jnp.int32)
        key = jnp.left_shift(q18 - 2**17, 14) | j
        t = jnp.where(j == qidx, INTMAX, key)
        pool = pools[v % NSETS]
        for l in range(NLVL):
            lo = jnp.minimum(pool[l], t)
            t = jnp.maximum(pool[l], t)
            pool[l] = lo
    merged = pools[0] + [jnp.full((8, 128), INTMAX, jnp.int32)
                         for _ in range(MLVL - NLVL)]
    for s in range(1, NSETS):
        for l2 in range(NLVL):
            t = pools[s][l2]
            for l in range(MLVL):
                lo = jnp.minimum(merged[l], t)
                t = jnp.maximum(merged[l], t)
                merged[l] = lo
    return merged


def _knn_body(posl_ref, posq_ref, out_ref):
    i = pl.program_id(0)
    lane = lax.broadcasted_iota(jnp.int32, (8, 128), 1)
    gpools = []
    for g in range(QG):
        qx = posq_ref[g * 8:(g + 1) * 8, 0:1]
        qy = posq_ref[g * 8:(g + 1) * 8, 1:2]
        qz = posq_ref[g * 8:(g + 1) * 8, 2:3]
        qidx = (i * QG + g) * 8 + lax.broadcasted_iota(jnp.int32, (8, 1), 0)
        gpools.append(_knn_phase1(posl_ref, qx, qy, qz, qidx, lane))

    idxs = [[] for _ in range(QG)]
    for _ in range(K_NB):
        for g in range(QG):
            pool = gpools[g]
            m = jnp.min(pool[0], axis=1, keepdims=True)
            idxs[g].append(jnp.bitwise_and(m, 16383))
            c = pool[0] == m
            for l in range(MLVL - 1):
                pool[l] = jnp.where(c, pool[l + 1], pool[l])
            pool[MLVL - 1] = jnp.where(c, INTMAX, pool[MLVL - 1])
    out = jnp.concatenate(
        [jnp.concatenate(idxs[g], axis=1) for g in range(QG)], axis=0)
    out_ref[...] = out


def _knn(posl, posq):
    return pl.pallas_call(
        _knn_body,
        grid=(NP // (8 * QG),),
        in_specs=[
            pl.BlockSpec((3 * NCHUNK, 8, 128), lambda i: (0, 0, 0)),
            pl.BlockSpec((8 * QG, 3), lambda i: (i, 0)),
        ],
        out_specs=pl.BlockSpec((8 * QG, K_NB), lambda i: (i, 0)),
        out_shape=jax.ShapeDtypeStruct((NP, K_NB), jnp.int32),
    )(posl, posq)


def _sqrt_sc(a):
    b = lax.bitcast_convert_type(a, jnp.int32)
    y = lax.bitcast_convert_type(
        jnp.right_shift(b, 1) + jnp.int32(0x1FBD1DF5), jnp.float32)
    for _ in range(3):
        y = 0.5 * (y + a / y)
    return y


def _geom_body(posx_h, posy_h, posz_h, nbr_h, s8_h, gidx_h,
               px_v, py_v, pz_v, nbr_v, s8_v, gi_v):
    wid = lax.axis_index("s") * 2 + lax.axis_index("c")
    base = wid * QPW
    pltpu.sync_copy(posx_h, px_v)
    pltpu.sync_copy(posy_h, py_v)
    pltpu.sync_copy(posz_h, pz_v)
    pltpu.sync_copy(nbr_h.at[pl.ds(base * K_NB, QPW * K_NB)], nbr_v)

    cq = 16

    def chunk(ci, _):
        def one_q(iq, _):
            q = ci * cq + iq
            qsplat = jnp.full((16,), base + q, jnp.int32)
            qx = plsc.load_gather(px_v, [qsplat])
            qy = plsc.load_gather(py_v, [qsplat])
            qz = plsc.load_gather(pz_v, [qsplat])
            jdx = nbr_v[pl.ds(q * K_NB, K_NB)]
            pjx = plsc.load_gather(px_v, [jdx])
            pjy = plsc.load_gather(py_v, [jdx])
            pjz = plsc.load_gather(pz_v, [jdx])
            rx = (pjx - qx) * (1.0 / RADIUS)
            ry = (pjy - qy) * (1.0 / RADIUS)
            rz = (pjz - qz) * (1.0 / RADIUS)
            r2 = rx * rx + ry * ry + rz * rz
            r2c = jnp.minimum(jnp.maximum(r2, 0.0), 1.0)
            om = 1.0 - r2c
            win = om * om * om
            win = jnp.where(r2 <= 1.0, win, 0.0)
            norm2 = _sqrt_sc(jnp.maximum(r2, 1e-12))
            ninf = jnp.maximum(
                jnp.maximum(jnp.abs(rx), jnp.abs(ry)),
                jnp.maximum(jnp.abs(rz), 1e-12))
            s = norm2 / ninf
            ux = jnp.clip((jnp.clip(rx * s, -1.0, 1.0) + 1.0) * 1.5, 0.0, 3.0)
            uy = jnp.clip((jnp.clip(ry * s, -1.0, 1.0) + 1.0) * 1.5, 0.0, 3.0)
            uz = jnp.clip((jnp.clip(rz * s, -1.0, 1.0) + 1.0) * 1.5, 0.0, 3.0)
            ix = jnp.minimum(ux.astype(jnp.int32), 2)
            iy = jnp.minimum(uy.astype(jnp.int32), 2)
            iz = jnp.minimum(uz.astype(jnp.int32), 2)
            fx = ux - ix.astype(jnp.float32)
            fy = uy - iy.astype(jnp.float32)
            fz = uz - iz.astype(jnp.float32)
            j64 = jdx * 64
            gb = ix * 16 + iy * 4 + iz
            for c in range(8):
                cx, cy, cz = (c >> 2) & 1, (c >> 1) & 1, c & 1
                wx = fx if cx else 1.0 - fx
                wy = fy if cy else 1.0 - fy
                wz = fz if cz else 1.0 - fz
                w = win * wx * wy * wz
                g = gb + (cx * 16 + cy * 4 + cz)
                off = iq * 128 + c * 16
                s8_v[pl.ds(off, 16)] = w
                gi_v[pl.ds(off, 16)] = j64 + g
            return 0

        lax.fori_loop(0, cq, one_q, 0)
        hb = (base + ci * cq) * 128
        pltpu.sync_copy(s8_v, s8_h.at[pl.ds(hb, cq * 128)])
        pltpu.sync_copy(gi_v, gidx_h.at[pl.ds(hb, cq * 128)])
        return 0

    lax.fori_loop(0, QPW // cq, chunk, 0)


def _geometry(posx, posy, posz, nbr):
    mesh = plsc.VectorSubcoreMesh(core_axis_name="c", subcore_axis_name="s")
    f = pl.kernel(
        _geom_body,
        out_type=(
            jax.ShapeDtypeStruct((NP * 128,), jnp.float32),
            jax.ShapeDtypeStruct((NP * 128,), jnp.int32),
        ),
        mesh=mesh,
        compiler_params=_SC_PARAMS,
        scratch_types=[
            pltpu.VMEM((NP,), jnp.float32),
            pltpu.VMEM((NP,), jnp.float32),
            pltpu.VMEM((NP,), jnp.float32),
            pltpu.VMEM((QPW * K_NB,), jnp.int32),
            pltpu.VMEM((16 * 128,), jnp.float32),
            pltpu.VMEM((16 * 128,), jnp.int32),
        ],
    )
    return f(posx, posy, posz, nbr.reshape(NP * K_NB))


def _make_fma_body(co):
    nacc = co // 16
    cqs = 8

    def body(p_h, gidx_h, s8_h, out_h, idx_v, s_v, rows_v, acc_v, gsem):
        wid = lax.axis_index("s") * 2 + lax.axis_index("c")
        base = wid * QPW

        def chunk(ci, _):
            q0 = base + ci * cqs
            pltpu.sync_copy(gidx_h.at[pl.ds(q0 * 128, cqs * 128)], idx_v)
            pltpu.sync_copy(s8_h.at[pl.ds(q0 * 128, cqs * 128)], s_v)
            cps = [
                pltpu.async_copy(
                    p_h.at[idx_v.at[pl.ds(j * 128, 128)]],
                    rows_v.at[pl.ds(j * 128, 128)], gsem)
                for j in range(cqs)
            ]
            for cp in cps:
                cp.wait()

            lane16 = lax.broadcasted_iota(jnp.int32, (16,), 0)

            def one_q(j, _):
                def grp(g, accs):
                    sbase = j * 128 + g * 16
                    for t in range(16):
                        sb = plsc.load_gather(
                            s_v, [jnp.full((16,), sbase + t, jnp.int32)])
                        row = jnp.full((16,), sbase + t, jnp.int32)
                        accs = tuple(
                            accs[u] + sb * plsc.load_gather(
                                rows_v, [row, u * 16 + lane16])
                            for u in range(nacc))
                    return accs

                accs = lax.fori_loop(
                    0, 8, grp,
                    tuple(jnp.zeros((16,), jnp.float32) for _ in range(nacc)))
                for u in range(nacc):
                    acc_v[pl.ds(j * co + u * 16, 16)] = accs[u]
                return 0

            lax.fori_loop(0, cqs, one_q, 0)
            pltpu.sync_copy(
                acc_v, out_h.at[pl.ds(q0 * co, cqs * co)])
            return 0

        lax.fori_loop(0, QPW // cqs, chunk, 0)

    return body, cqs


def _sc_fma(p_rows, gidx, s8, co):
    body, cqs = _make_fma_body(co)
    mesh = plsc.VectorSubcoreMesh(core_axis_name="c", subcore_axis_name="s")
    f = pl.kernel(
        body,
        out_type=jax.ShapeDtypeStruct((NP * co,), jnp.float32),
        mesh=mesh,
        compiler_params=_SC_PARAMS,
        scratch_types=[
            pltpu.VMEM((cqs * 128,), jnp.int32),
            pltpu.VMEM((cqs * 128,), jnp.float32),
            pltpu.VMEM((cqs * 128, co), jnp.float32),
            pltpu.VMEM((cqs * co,), jnp.float32),
            pltpu.SemaphoreType.DMA,
        ],
    )
    return f(p_rows, gidx, s8).reshape(NP, co)


def _mm_body(x_ref, w_ref, b_ref, o1_ref, o2_ref, *, n1):
    res = jnp.dot(x_ref[...], w_ref[...], preferred_element_type=jnp.float32)
    o1_ref[...] = res[:, :n1]
    o2_ref[...] = res[:, n1:] + b_ref[...]


def _mm(x, w, bias2, n1, n2, bm=256):
    m, k = x.shape
    return pl.pallas_call(
        functools.partial(_mm_body, n1=n1),
        grid=(m // bm,),
        in_specs=[
            pl.BlockSpec((bm, k), lambda i: (i, 0)),
            pl.BlockSpec((k, n1 + n2), lambda i: (0, 0)),
            pl.BlockSpec((1, n2), lambda i: (0, 0)),
        ],
        out_specs=[
            pl.BlockSpec((bm, n1), lambda i: (i, 0)),
            pl.BlockSpec((bm, n2), lambda i: (i, 0)),
        ],
        out_shape=[
            jax.ShapeDtypeStruct((m, n1), jnp.float32),
            jax.ShapeDtypeStruct((m, n2), jnp.float32),
        ],
    )(x, w, bias2)


def _combine_body(c_ref, d_ref, r_ref, o_ref):
    o_ref[...] = jnp.maximum(c_ref[...] + d_ref[...] + r_ref[...], 0.0)


def _combine_body2(c_ref, d_ref, o_ref):
    o_ref[...] = jnp.maximum(c_ref[...] + d_ref[...], 0.0)


def _combine(c, d, res=None, bm=512):
    m, n = c.shape
    if res is None:
        return pl.pallas_call(
            _combine_body2,
            grid=(m // bm,),
            in_specs=[pl.BlockSpec((bm, n), lambda i: (i, 0))] * 2,
            out_specs=pl.BlockSpec((bm, n), lambda i: (i, 0)),
            out_shape=jax.ShapeDtypeStruct((m, n), jnp.float32),
        )(c, d)
    return pl.pallas_call(
        _combine_body,
        grid=(m // bm,),
        in_specs=[pl.BlockSpec((bm, n), lambda i: (i, 0))] * 3,
        out_specs=pl.BlockSpec((bm, n), lambda i: (i, 0)),
        out_shape=jax.ShapeDtypeStruct((m, n), jnp.float32),
    )(c, d, res)


def _prep_w(Wc, Wd, bd, cin_pad, co_pad):
    ks3, cin, co = Wc.shape
    wc = jnp.zeros((ks3, cin_pad, co_pad), jnp.float32).at[:, :cin, :co].set(Wc)
    wcf = wc.transpose(1, 0, 2).reshape(cin_pad, ks3 * co_pad)
    wd = jnp.zeros((cin_pad, co_pad), jnp.float32).at[:cin, :co].set(Wd)
    w = jnp.concatenate([wcf, wd], axis=1)
    bias = jnp.zeros((1, co_pad), jnp.float32).at[0, :co].set(bd)
    return w, bias


def kernel(pos, vel, mass, extra_feats,
           Wc0, Wd0, bd0, Wc1, Wd1, bd1, Wc2, Wd2, bd2, Wc3, Wd3, bd3):
    posp = jnp.full((NP, 3), 1e4, jnp.float32).at[:NPTS].set(pos)
    posl = posp.T.reshape(3, NCHUNK, 1, 128).reshape(3 * NCHUNK, 1, 128)
    posl = jnp.broadcast_to(posl, (3 * NCHUNK, 8, 128))

    nbr = _knn(posl, posp)

    posx, posy, posz = posp[:, 0], posp[:, 1], posp[:, 2]
    s8, gidx = _geometry(posx, posy, posz, nbr)

    feats = jnp.concatenate([mass[:, None], vel, extra_feats], axis=-1)
    x = jnp.zeros((NP, 16), jnp.float32).at[:NPTS, :13].set(feats)

    def layer(x, Wc, Wd, bd, cin_pad, co_pad):
        w, bias = _prep_w(Wc, Wd, bd, cin_pad, co_pad)
        p, d = _mm(x, w, bias, 64 * co_pad, co_pad)
        p_rows = p.reshape(NP * 64, co_pad)
        c = _sc_fma(p_rows, gidx, s8, co_pad)
        return c, d

    c0, d0 = layer(x, Wc0, Wd0, bd0, 16, 32)
    x1 = jnp.concatenate([c0, d0], axis=1)
    c1, d1 = layer(x1, Wc1, Wd1, bd1, 64, 64)
    x2 = _combine(c1, d1, x1)
    c2, d2 = layer(x2, Wc2, Wd2, bd2, 64, 64)
    x3 = _combine(c2, d2, x2)
    c3, d3 = layer(x3, Wc3, Wd3, bd3, 64, 16)
    out = _combine(c3, d3)
    return out[:NPTS, :3]


# speedup vs baseline: 2.0923x; 1.1595x over previous
import functools

import jax
import jax.numpy as jnp
from jax import lax
from jax.experimental import pallas as pl
from jax.experimental.pallas import tpu as pltpu
from jax.experimental.pallas import tpu_sc as plsc

NPTS = 10000
NP = 10240
K_NB = 16
RADIUS = 1.5
KS = 4
NCHUNK = NP // 128
INTMAX = 2**31 - 1
D2CLIP = 2.26
KEYSCALE = float((2**18 - 1) / D2CLIP)
NW = 32
QPW = NP // NW
_SC_PARAMS = pltpu.CompilerParams(
    needs_layout_passes=False, use_tc_tiling_on_sc=False)


NLVL = 4
NSETS = 3


QG = 8
MLVL = 6


def _knn_phase1(posl_ref, qx, qy, qz, qidx, lane):
    pools = [[jnp.full((8, 128), INTMAX, jnp.int32) for _ in range(NLVL)]
             for _ in range(NSETS)]
    for v in range(NCHUNK):
        px = posl_ref[v]
        py = posl_ref[NCHUNK + v]
        pz = posl_ref[2 * NCHUNK + v]
        dx = qx - px
        dy = qy - py
        dz = qz - pz
        d2 = dx * dx + dy * dy + dz * dz
        j = v * 128 + lane
        q18 = (jnp.minimum(d2, D2CLIP) * KEYSCALE).astype(jnp.int32)
        key = jnp.left_shift(q18 - 2**17, 14) | j
        t = jnp.where(j == qidx, INTMAX, key)
        pool = pools[v % NSETS]
        for l in range(NLVL):
            lo = jnp.minimum(pool[l], t)
            t = jnp.maximum(pool[l], t)
            pool[l] = lo
    merged = pools[0] + [jnp.full((8, 128), INTMAX, jnp.int32)
                         for _ in range(MLVL - NLVL)]
    for s in range(1, NSETS):
        for l2 in range(NLVL):
            t = pools[s][l2]
            for l in range(MLVL):
                lo = jnp.minimum(merged[l], t)
                t = jnp.maximum(merged[l], t)
                merged[l] = lo
    return merged


def _knn_body(posl_ref, posq_ref, out_ref):
    i = pl.program_id(0)
    lane = lax.broadcasted_iota(jnp.int32, (8, 128), 1)
    gpools = []
    for g in range(QG):
        qx = posq_ref[g * 8:(g + 1) * 8, 0:1]
        qy = posq_ref[g * 8:(g + 1) * 8, 1:2]
        qz = posq_ref[g * 8:(g + 1) * 8, 2:3]
        qidx = (i * QG + g) * 8 + lax.broadcasted_iota(jnp.int32, (8, 1), 0)
        gpools.append(_knn_phase1(posl_ref, qx, qy, qz, qidx, lane))

    idxs = [[] for _ in range(QG)]
    for _ in range(K_NB):
        for g in range(QG):
            pool = gpools[g]
            m = jnp.min(pool[0], axis=1, keepdims=True)
            idxs[g].append(jnp.bitwise_and(m, 16383))
            c = pool[0] == m
            for l in range(MLVL - 1):
                pool[l] = jnp.where(c, pool[l + 1], pool[l])
            pool[MLVL - 1] = jnp.where(c, INTMAX, pool[MLVL - 1])
    out = jnp.concatenate(
        [jnp.concatenate(idxs[g], axis=1) for g in range(QG)], axis=0)
    out_ref[...] = out


def _knn(posl, posq):
    return pl.pallas_call(
        _knn_body,
        grid=(NP // (8 * QG),),
        in_specs=[
            pl.BlockSpec((3 * NCHUNK, 8, 128), lambda i: (0, 0, 0)),
            pl.BlockSpec((8 * QG, 3), lambda i: (i, 0)),
        ],
        out_specs=pl.BlockSpec((8 * QG, K_NB), lambda i: (i, 0)),
        out_shape=jax.ShapeDtypeStruct((NP, K_NB), jnp.int32),
    )(posl, posq)


def _sqrt_sc(a):
    b = lax.bitcast_convert_type(a, jnp.int32)
    y = lax.bitcast_convert_type(
        jnp.right_shift(b, 1) + jnp.int32(0x1FBD1DF5), jnp.float32)
    for _ in range(3):
        y = 0.5 * (y + a / y)
    return y


def _geom_body(posx_h, posy_h, posz_h, nbr_h, s8_h, gidx_h,
               px_v, py_v, pz_v, nbr_v, s8_v, gi_v):
    wid = lax.axis_index("s") * 2 + lax.axis_index("c")
    base = wid * QPW
    pltpu.sync_copy(posx_h, px_v)
    pltpu.sync_copy(posy_h, py_v)
    pltpu.sync_copy(posz_h, pz_v)
    pltpu.sync_copy(nbr_h.at[pl.ds(base * K_NB, QPW * K_NB)], nbr_v)

    cq = 16

    def chunk(ci, _):
        def one_q(iq, _):
            q = ci * cq + iq
            qsplat = jnp.full((16,), base + q, jnp.int32)
            qx = plsc.load_gather(px_v, [qsplat])
            qy = plsc.load_gather(py_v, [qsplat])
            qz = plsc.load_gather(pz_v, [qsplat])
            jdx = nbr_v[pl.ds(q * K_NB, K_NB)]
            pjx = plsc.load_gather(px_v, [jdx])
            pjy = plsc.load_gather(py_v, [jdx])
            pjz = plsc.load_gather(pz_v, [jdx])
            rx = (pjx - qx) * (1.0 / RADIUS)
            ry = (pjy - qy) * (1.0 / RADIUS)
            rz = (pjz - qz) * (1.0 / RADIUS)
            r2 = rx * rx + ry * ry + rz * rz
            r2c = jnp.minimum(jnp.maximum(r2, 0.0), 1.0)
            om = 1.0 - r2c
            win = om * om * om
            win = jnp.where(r2 <= 1.0, win, 0.0)
            norm2 = _sqrt_sc(jnp.maximum(r2, 1e-12))
            ninf = jnp.maximum(
                jnp.maximum(jnp.abs(rx), jnp.abs(ry)),
                jnp.maximum(jnp.abs(rz), 1e-12))
            s = norm2 / ninf
            ux = jnp.clip((jnp.clip(rx * s, -1.0, 1.0) + 1.0) * 1.5, 0.0, 3.0)
            uy = jnp.clip((jnp.clip(ry * s, -1.0, 1.0) + 1.0) * 1.5, 0.0, 3.0)
            uz = jnp.clip((jnp.clip(rz * s, -1.0, 1.0) + 1.0) * 1.5, 0.0, 3.0)
            ix = jnp.minimum(ux.astype(jnp.int32), 2)
            iy = jnp.minimum(uy.astype(jnp.int32), 2)
            iz = jnp.minimum(uz.astype(jnp.int32), 2)
            fx = ux - ix.astype(jnp.float32)
            fy = uy - iy.astype(jnp.float32)
            fz = uz - iz.astype(jnp.float32)
            j64 = jdx * 64
            gb = ix * 16 + iy * 4 + iz
            for c in range(8):
                cx, cy, cz = (c >> 2) & 1, (c >> 1) & 1, c & 1
                wx = fx if cx else 1.0 - fx
                wy = fy if cy else 1.0 - fy
                wz = fz if cz else 1.0 - fz
                w = win * wx * wy * wz
                g = gb + (cx * 16 + cy * 4 + cz)
                off = iq * 128 + c * 16
                s8_v[pl.ds(off, 16)] = w
                gi_v[pl.ds(off, 16)] = j64 + g
            return 0

        lax.fori_loop(0, cq, one_q, 0)
        hb = (base + ci * cq) * 128
        pltpu.sync_copy(s8_v, s8_h.at[pl.ds(hb, cq * 128)])
        pltpu.sync_copy(gi_v, gidx_h.at[pl.ds(hb, cq * 128)])
        return 0

    lax.fori_loop(0, QPW // cq, chunk, 0)


def _geometry(posx, posy, posz, nbr):
    mesh = plsc.VectorSubcoreMesh(core_axis_name="c", subcore_axis_name="s")
    f = pl.kernel(
        _geom_body,
        out_type=(
            jax.ShapeDtypeStruct((NP * 128,), jnp.float32),
            jax.ShapeDtypeStruct((NP * 128,), jnp.int32),
        ),
        mesh=mesh,
        compiler_params=_SC_PARAMS,
        scratch_types=[
            pltpu.VMEM((NP,), jnp.float32),
            pltpu.VMEM((NP,), jnp.float32),
            pltpu.VMEM((NP,), jnp.float32),
            pltpu.VMEM((QPW * K_NB,), jnp.int32),
            pltpu.VMEM((16 * 128,), jnp.float32),
            pltpu.VMEM((16 * 128,), jnp.int32),
        ],
    )
    return f(posx, posy, posz, nbr.reshape(NP * K_NB))


def _make_fma_body(co):
    nacc = co // 16
    cqs = 8

    def body(p_h, gidx_h, s8_h, out_h, idx_v, s_v, rows_v, acc_v, gsem):
        wid = lax.axis_index("s") * 2 + lax.axis_index("c")
        base = wid * QPW

        def chunk(ci, _):
            q0 = base + ci * cqs
            pltpu.sync_copy(gidx_h.at[pl.ds(q0 * 128, cqs * 128)], idx_v)
            pltpu.sync_copy(s8_h.at[pl.ds(q0 * 128, cqs * 128)], s_v)
            cps = [
                pltpu.async_copy(
                    p_h.at[idx_v.at[pl.ds(j * 128, 128)]],
                    rows_v.at[pl.ds(j * 128, 128)], gsem)
                for j in range(cqs)
            ]
            for cp in cps:
                cp.wait()

            lane16 = lax.broadcasted_iota(jnp.int32, (16,), 0)

            def one_q(j, _):
                def grp(g, accs):
                    sbase = j * 128 + g * 16
                    for t in range(16):
                        sb = plsc.load_gather(
                            s_v, [jnp.full((16,), sbase + t, jnp.int32)])
                        row = jnp.full((16,), sbase + t, jnp.int32)
                        accs = tuple(
                            accs[u] + sb * plsc.load_gather(
                                rows_v, [row, u * 16 + lane16])
                            for u in range(nacc))
                    return accs

                accs = lax.fori_loop(
                    0, 8, grp,
                    tuple(jnp.zeros((16,), jnp.float32) for _ in range(nacc)))
                for u in range(nacc):
                    acc_v[pl.ds(j * co + u * 16, 16)] = accs[u]
                return 0

            lax.fori_loop(0, cqs, one_q, 0)
            pltpu.sync_copy(
                acc_v, out_h.at[pl.ds(q0 * co, cqs * co)])
            return 0

        lax.fori_loop(0, QPW // cqs, chunk, 0)

    return body, cqs


def _sc_fma(p_rows, gidx, s8, co):
    body, cqs = _make_fma_body(co)
    mesh = plsc.VectorSubcoreMesh(core_axis_name="c", subcore_axis_name="s")
    f = pl.kernel(
        body,
        out_type=jax.ShapeDtypeStruct((NP * co,), jnp.float32),
        mesh=mesh,
        compiler_params=_SC_PARAMS,
        scratch_types=[
            pltpu.VMEM((cqs * 128,), jnp.int32),
            pltpu.VMEM((cqs * 128,), jnp.float32),
            pltpu.VMEM((cqs * 128, co), jnp.float32),
            pltpu.VMEM((cqs * co,), jnp.float32),
            pltpu.SemaphoreType.DMA,
        ],
    )
    return f(p_rows, gidx, s8).reshape(NP, co)


def _mm_body(x_ref, w_ref, b_ref, o1_ref, o2_ref, *, n1):
    res = jnp.dot(x_ref[...], w_ref[...], preferred_element_type=jnp.float32)
    o1_ref[...] = res[:, :n1]
    o2_ref[...] = res[:, n1:] + b_ref[...]


def _mm(x, w, bias2, n1, n2, bm=256):
    m, k = x.shape
    return pl.pallas_call(
        functools.partial(_mm_body, n1=n1),
        grid=(m // bm,),
        in_specs=[
            pl.BlockSpec((bm, k), lambda i: (i, 0)),
            pl.BlockSpec((k, n1 + n2), lambda i: (0, 0)),
            pl.BlockSpec((1, n2), lambda i: (0, 0)),
        ],
        out_specs=[
            pl.BlockSpec((bm, n1), lambda i: (i, 0)),
            pl.BlockSpec((bm, n2), lambda i: (i, 0)),
        ],
        out_shape=[
            jax.ShapeDtypeStruct((m, n1), jnp.float32),
            jax.ShapeDtypeStruct((m, n2), jnp.float32),
        ],
    )(x, w, bias2)


def _combine_body(c_ref, d_ref, r_ref, o_ref):
    o_ref[...] = jnp.maximum(c_ref[...] + d_ref[...] + r_ref[...], 0.0)


def _combine_body2(c_ref, d_ref, o_ref):
    o_ref[...] = jnp.maximum(c_ref[...] + d_ref[...], 0.0)


def _combine(c, d, res=None, bm=512):
    m, n = c.shape
    if res is None:
        return pl.pallas_call(
            _combine_body2,
            grid=(m // bm,),
            in_specs=[pl.BlockSpec((bm, n), lambda i: (i, 0))] * 2,
            out_specs=pl.BlockSpec((bm, n), lambda i: (i, 0)),
            out_shape=jax.ShapeDtypeStruct((m, n), jnp.float32),
        )(c, d)
    return pl.pallas_call(
        _combine_body,
        grid=(m // bm,),
        in_specs=[pl.BlockSpec((bm, n), lambda i: (i, 0))] * 3,
        out_specs=pl.BlockSpec((bm, n), lambda i: (i, 0)),
        out_shape=jax.ShapeDtypeStruct((m, n), jnp.float32),
    )(c, d, res)


def _prep_w(Wc, Wd, bd, cin_pad, co_pad):
    ks3, cin, co = Wc.shape
    wc = jnp.zeros((ks3, cin_pad, co_pad), jnp.float32).at[:, :cin, :co].set(Wc)
    wcf = wc.transpose(1, 0, 2).reshape(cin_pad, ks3 * co_pad)
    wd = jnp.zeros((cin_pad, co_pad), jnp.float32).at[:cin, :co].set(Wd)
    w = jnp.concatenate([wcf, wd], axis=1)
    bias = jnp.zeros((1, co_pad), jnp.float32).at[0, :co].set(bd)
    return w, bias


def kernel(pos, vel, mass, extra_feats,
           Wc0, Wd0, bd0, Wc1, Wd1, bd1, Wc2, Wd2, bd2, Wc3, Wd3, bd3):
    posp = jnp.full((NP, 3), 1e4, jnp.float32).at[:NPTS].set(pos)
    posl = posp.T.reshape(3, NCHUNK, 1, 128).reshape(3 * NCHUNK, 1, 128)
    posl = jnp.broadcast_to(posl, (3 * NCHUNK, 8, 128))

    nbr = _knn(posl, posp)

    posx, posy, posz = posp[:, 0], posp[:, 1], posp[:, 2]
    s8, gidx = _geometry(posx, posy, posz, nbr)

    feats = jnp.concatenate([mass[:, None], vel, extra_feats], axis=-1)
    x = jnp.zeros((NP, 16), jnp.float32).at[:NPTS, :13].set(feats)

    def layer(x, Wc, Wd, bd, cin_pad, co_pad):
        w, bias = _prep_w(Wc, Wd, bd, cin_pad, co_pad)
        p, d = _mm(x, w, bias, 64 * co_pad, co_pad)
        p_rows = p.reshape(NP * 64, co_pad)
        c = _sc_fma(p_rows, gidx, s8, co_pad)
        return c, d

    c0, d0 = layer(x, Wc0, Wd0, bd0, 16, 32)
    x1 = jnp.concatenate([c0, d0], axis=1)
    c1, d1 = layer(x1, Wc1, Wd1, bd1, 64, 64)
    x2 = _combine(c1, d1, x1)
    c2, d2 = layer(x2, Wc2, Wd2, bd2, 64, 64)
    x3 = _combine(c2, d2, x2)
    c3, d3 = layer(x3, Wc3, Wd3, bd3, 64, 16)
    out = _combine(c3, d3)
    return out[:NPTS, :3]


# speedup vs baseline: 2.3812x; 1.1381x over previous
import functools

import jax
import jax.numpy as jnp
from jax import lax
from jax.experimental import pallas as pl
from jax.experimental.pallas import tpu as pltpu
from jax.experimental.pallas import tpu_sc as plsc

NPTS = 10000
NP = 10240
K_NB = 16
RADIUS = 1.5
KS = 4
NCHUNK = NP // 128
INTMAX = 2**31 - 1
D2CLIP = 2.26
KEYSCALE = float((2**18 - 1) / D2CLIP)
NW = 32
QPW = NP // NW
_SC_PARAMS = pltpu.CompilerParams(
    needs_layout_passes=False, use_tc_tiling_on_sc=False)


NLVL = 4
NSETS = 3


QG = 8
MLVL = 6


def _knn_phase1(posl_ref, qx, qy, qz, qidx, lane):
    pools = [[jnp.full((8, 128), INTMAX, jnp.int32) for _ in range(NLVL)]
             for _ in range(NSETS)]
    for v in range(NCHUNK):
        px = posl_ref[v]
        py = posl_ref[NCHUNK + v]
        pz = posl_ref[2 * NCHUNK + v]
        dx = qx - px
        dy = qy - py
        dz = qz - pz
        d2 = dx * dx + dy * dy + dz * dz
        j = v * 128 + lane
        q18 = (jnp.minimum(d2, D2CLIP) * KEYSCALE).astype(jnp.int32)
        key = jnp.left_shift(q18 - 2**17, 14) | j
        t = jnp.where(j == qidx, INTMAX, key)
        pool = pools[v % NSETS]
        for l in range(NLVL):
            lo = jnp.minimum(pool[l], t)
            t = jnp.maximum(pool[l], t)
            pool[l] = lo
    merged = pools[0] + [jnp.full((8, 128), INTMAX, jnp.int32)
                         for _ in range(MLVL - NLVL)]
    for s in range(1, NSETS):
        for l2 in range(NLVL):
            t = pools[s][l2]
            for l in range(MLVL):
                lo = jnp.minimum(merged[l], t)
                t = jnp.maximum(merged[l], t)
                merged[l] = lo
    return merged


def _knn_body(posl_ref, posq_ref, out_ref):
    i = pl.program_id(0)
    lane = lax.broadcasted_iota(jnp.int32, (8, 128), 1)
    gpools = []
    for g in range(QG):
        qx = posq_ref[g * 8:(g + 1) * 8, 0:1]
        qy = posq_ref[g * 8:(g + 1) * 8, 1:2]
        qz = posq_ref[g * 8:(g + 1) * 8, 2:3]
        qidx = (i * QG + g) * 8 + lax.broadcasted_iota(jnp.int32, (8, 1), 0)
        gpools.append(_knn_phase1(posl_ref, qx, qy, qz, qidx, lane))

    idxs = [[] for _ in range(QG)]
    for _ in range(K_NB):
        for g in range(QG):
            pool = gpools[g]
            m = jnp.min(pool[0], axis=1, keepdims=True)
            idxs[g].append(jnp.bitwise_and(m, 16383))
            c = pool[0] == m
            for l in range(MLVL - 1):
                pool[l] = jnp.where(c, pool[l + 1], pool[l])
            pool[MLVL - 1] = jnp.where(c, INTMAX, pool[MLVL - 1])
    out = jnp.concatenate(
        [jnp.concatenate(idxs[g], axis=1) for g in range(QG)], axis=0)
    out_ref[...] = out


def _knn(posl, posq):
    return pl.pallas_call(
        _knn_body,
        grid=(NP // (8 * QG),),
        in_specs=[
            pl.BlockSpec((3 * NCHUNK, 8, 128), lambda i: (0, 0, 0)),
            pl.BlockSpec((8 * QG, 3), lambda i: (i, 0)),
        ],
        out_specs=pl.BlockSpec((8 * QG, K_NB), lambda i: (i, 0)),
        out_shape=jax.ShapeDtypeStruct((NP, K_NB), jnp.int32),
    )(posl, posq)


def _sqrt_sc(a):
    b = lax.bitcast_convert_type(a, jnp.int32)
    y = lax.bitcast_convert_type(
        jnp.right_shift(b, 1) + jnp.int32(0x1FBD1DF5), jnp.float32)
    for _ in range(3):
        y = 0.5 * (y + a / y)
    return y


def _geom_body(posx_h, posy_h, posz_h, nbr_h, s8_h, gidx_h,
               px_v, py_v, pz_v, nbr_v, s8_v, gi_v):
    wid = lax.axis_index("s") * 2 + lax.axis_index("c")
    base = wid * QPW
    pltpu.sync_copy(posx_h, px_v)
    pltpu.sync_copy(posy_h, py_v)
    pltpu.sync_copy(posz_h, pz_v)
    pltpu.sync_copy(nbr_h.at[pl.ds(base * K_NB, QPW * K_NB)], nbr_v)

    cq = 16

    def chunk(ci, _):
        def one_q(iq, _):
            q = ci * cq + iq
            qsplat = jnp.full((16,), base + q, jnp.int32)
            qx = plsc.load_gather(px_v, [qsplat])
            qy = plsc.load_gather(py_v, [qsplat])
            qz = plsc.load_gather(pz_v, [qsplat])
            jdx = nbr_v[pl.ds(q * K_NB, K_NB)]
            pjx = plsc.load_gather(px_v, [jdx])
            pjy = plsc.load_gather(py_v, [jdx])
            pjz = plsc.load_gather(pz_v, [jdx])
            rx = (pjx - qx) * (1.0 / RADIUS)
            ry = (pjy - qy) * (1.0 / RADIUS)
            rz = (pjz - qz) * (1.0 / RADIUS)
            r2 = rx * rx + ry * ry + rz * rz
            r2c = jnp.minimum(jnp.maximum(r2, 0.0), 1.0)
            om = 1.0 - r2c
            win = om * om * om
            win = jnp.where(r2 <= 1.0, win, 0.0)
            norm2 = _sqrt_sc(jnp.maximum(r2, 1e-12))
            ninf = jnp.maximum(
                jnp.maximum(jnp.abs(rx), jnp.abs(ry)),
                jnp.maximum(jnp.abs(rz), 1e-12))
            s = norm2 / ninf
            ux = jnp.clip((jnp.clip(rx * s, -1.0, 1.0) + 1.0) * 1.5, 0.0, 3.0)
            uy = jnp.clip((jnp.clip(ry * s, -1.0, 1.0) + 1.0) * 1.5, 0.0, 3.0)
            uz = jnp.clip((jnp.clip(rz * s, -1.0, 1.0) + 1.0) * 1.5, 0.0, 3.0)
            ix = jnp.minimum(ux.astype(jnp.int32), 2)
            iy = jnp.minimum(uy.astype(jnp.int32), 2)
            iz = jnp.minimum(uz.astype(jnp.int32), 2)
            fx = ux - ix.astype(jnp.float32)
            fy = uy - iy.astype(jnp.float32)
            fz = uz - iz.astype(jnp.float32)
            j64 = jdx * 64
            gb = ix * 16 + iy * 4 + iz
            for c in range(8):
                cx, cy, cz = (c >> 2) & 1, (c >> 1) & 1, c & 1
                wx = fx if cx else 1.0 - fx
                wy = fy if cy else 1.0 - fy
                wz = fz if cz else 1.0 - fz
                w = win * wx * wy * wz
                g = gb + (cx * 16 + cy * 4 + cz)
                off = iq * 128 + c * 16
                s8_v[pl.ds(off, 16)] = w
                gi_v[pl.ds(off, 16)] = j64 + g
            return 0

        lax.fori_loop(0, cq, one_q, 0)
        hb = (base + ci * cq) * 128
        pltpu.sync_copy(s8_v, s8_h.at[pl.ds(hb, cq * 128)])
        pltpu.sync_copy(gi_v, gidx_h.at[pl.ds(hb, cq * 128)])
        return 0

    lax.fori_loop(0, QPW // cq, chunk, 0)


def _geometry(posx, posy, posz, nbr):
    mesh = plsc.VectorSubcoreMesh(core_axis_name="c", subcore_axis_name="s")
    f = pl.kernel(
        _geom_body,
        out_type=(
            jax.ShapeDtypeStruct((NP * 128,), jnp.float32),
            jax.ShapeDtypeStruct((NP * 128,), jnp.int32),
        ),
        mesh=mesh,
        compiler_params=_SC_PARAMS,
        scratch_types=[
            pltpu.VMEM((NP,), jnp.float32),
            pltpu.VMEM((NP,), jnp.float32),
            pltpu.VMEM((NP,), jnp.float32),
            pltpu.VMEM((QPW * K_NB,), jnp.int32),
            pltpu.VMEM((16 * 128,), jnp.float32),
            pltpu.VMEM((16 * 128,), jnp.int32),
        ],
    )
    return f(posx, posy, posz, nbr.reshape(NP * K_NB))


def _make_fma_body(co, cqs):
    nacc = co // 16
    nch = QPW // cqs

    def body(p_h, gidx_h, s8_h, out_h,
             idx0, idx1, s0, s1, rows0, rows1, acc_v, sem0, sem1):
        wid = lax.axis_index("s") * 2 + lax.axis_index("c")
        base = wid * QPW
        lane16 = lax.broadcasted_iota(jnp.int32, (16,), 0)

        def issue(ci, idx_v, s_v, rows_v, sem):
            q0 = base + ci * cqs
            pltpu.sync_copy(gidx_h.at[pl.ds(q0 * 128, cqs * 128)], idx_v)
            pltpu.sync_copy(s8_h.at[pl.ds(q0 * 128, cqs * 128)], s_v)
            for j in range(cqs):
                pltpu.async_copy(
                    p_h.at[idx_v.at[pl.ds(j * 128, 128)]],
                    rows_v.at[pl.ds(j * 128, 128)], sem)

        def drain(rows_v, sem):
            pltpu.make_async_copy(
                p_h.at[pl.ds(0, cqs * 128)], rows_v, sem).wait()

        def compute(ci, s_v, rows_v):
            q0 = base + ci * cqs

            def one_q(j, _):
                def grp(g, accs):
                    sbase = j * 128 + g * 16
                    for t in range(16):
                        sb = plsc.load_gather(
                            s_v, [jnp.full((16,), sbase + t, jnp.int32)])
                        row = jnp.full((16,), sbase + t, jnp.int32)
                        accs = tuple(
                            accs[u] + sb * plsc.load_gather(
                                rows_v, [row, u * 16 + lane16])
                            for u in range(nacc))
                    return accs

                accs = lax.fori_loop(
                    0, 8, grp,
                    tuple(jnp.zeros((16,), jnp.float32) for _ in range(nacc)))
                for u in range(nacc):
                    acc_v[pl.ds(j * co + u * 16, 16)] = accs[u]
                return 0

            lax.fori_loop(0, cqs, one_q, 0)
            pltpu.sync_copy(acc_v, out_h.at[pl.ds(q0 * co, cqs * co)])

        issue(0, idx0, s0, rows0, sem0)

        def pair(cp, _):
            c0 = 2 * cp
            issue(c0 + 1, idx1, s1, rows1, sem1)
            drain(rows0, sem0)
            compute(c0, s0, rows0)

            @pl.when(c0 + 2 < nch)
            def _():
                issue(c0 + 2, idx0, s0, rows0, sem0)

            drain(rows1, sem1)
            compute(c0 + 1, s1, rows1)
            return 0

        lax.fori_loop(0, nch // 2, pair, 0)

    return body


def _sc_fma(p_rows, gidx, s8, co):
    cqs = 256 // co
    body = _make_fma_body(co, cqs)
    mesh = plsc.VectorSubcoreMesh(core_axis_name="c", subcore_axis_name="s")
    f = pl.kernel(
        body,
        out_type=jax.ShapeDtypeStruct((NP * co,), jnp.float32),
        mesh=mesh,
        compiler_params=_SC_PARAMS,
        scratch_types=[
            pltpu.VMEM((cqs * 128,), jnp.int32),
            pltpu.VMEM((cqs * 128,), jnp.int32),
            pltpu.VMEM((cqs * 128,), jnp.float32),
            pltpu.VMEM((cqs * 128,), jnp.float32),
            pltpu.VMEM((cqs * 128, co), jnp.float32),
            pltpu.VMEM((cqs * 128, co), jnp.float32),
            pltpu.VMEM((cqs * co,), jnp.float32),
            pltpu.SemaphoreType.DMA,
            pltpu.SemaphoreType.DMA,
        ],
    )
    return f(p_rows, gidx, s8).reshape(NP, co)


def _mm_body(x_ref, w_ref, b_ref, o1_ref, o2_ref, *, n1):
    res = jnp.dot(x_ref[...], w_ref[...], preferred_element_type=jnp.float32)
    o1_ref[...] = res[:, :n1]
    o2_ref[...] = res[:, n1:] + b_ref[...]


def _mm(x, w, bias2, n1, n2, bm=256):
    m, k = x.shape
    return pl.pallas_call(
        functools.partial(_mm_body, n1=n1),
        grid=(m // bm,),
        in_specs=[
            pl.BlockSpec((bm, k), lambda i: (i, 0)),
            pl.BlockSpec((k, n1 + n2), lambda i: (0, 0)),
            pl.BlockSpec((1, n2), lambda i: (0, 0)),
        ],
        out_specs=[
            pl.BlockSpec((bm, n1), lambda i: (i, 0)),
            pl.BlockSpec((bm, n2), lambda i: (i, 0)),
        ],
        out_shape=[
            jax.ShapeDtypeStruct((m, n1), jnp.float32),
            jax.ShapeDtypeStruct((m, n2), jnp.float32),
        ],
    )(x, w, bias2)


def _combine_body(c_ref, d_ref, r_ref, o_ref):
    o_ref[...] = jnp.maximum(c_ref[...] + d_ref[...] + r_ref[...], 0.0)


def _combine_body2(c_ref, d_ref, o_ref):
    o_ref[...] = jnp.maximum(c_ref[...] + d_ref[...], 0.0)


def _combine(c, d, res=None, bm=512):
    m, n = c.shape
    if res is None:
        return pl.pallas_call(
            _combine_body2,
            grid=(m // bm,),
            in_specs=[pl.BlockSpec((bm, n), lambda i: (i, 0))] * 2,
            out_specs=pl.BlockSpec((bm, n), lambda i: (i, 0)),
            out_shape=jax.ShapeDtypeStruct((m, n), jnp.float32),
        )(c, d)
    return pl.pallas_call(
        _combine_body,
        grid=(m // bm,),
        in_specs=[pl.BlockSpec((bm, n), lambda i: (i, 0))] * 3,
        out_specs=pl.BlockSpec((bm, n), lambda i: (i, 0)),
        out_shape=jax.ShapeDtypeStruct((m, n), jnp.float32),
    )(c, d, res)


def _prep_w(Wc, Wd, bd, cin_pad, co_pad):
    ks3, cin, co = Wc.shape
    wc = jnp.zeros((ks3, cin_pad, co_pad), jnp.float32).at[:, :cin, :co].set(Wc)
    wcf = wc.transpose(1, 0, 2).reshape(cin_pad, ks3 * co_pad)
    wd = jnp.zeros((cin_pad, co_pad), jnp.float32).at[:cin, :co].set(Wd)
    w = jnp.concatenate([wcf, wd], axis=1)
    bias = jnp.zeros((1, co_pad), jnp.float32).at[0, :co].set(bd)
    return w, bias


def kernel(pos, vel, mass, extra_feats,
           Wc0, Wd0, bd0, Wc1, Wd1, bd1, Wc2, Wd2, bd2, Wc3, Wd3, bd3):
    posp = jnp.full((NP, 3), 1e4, jnp.float32).at[:NPTS].set(pos)
    posl = posp.T.reshape(3, NCHUNK, 1, 128).reshape(3 * NCHUNK, 1, 128)
    posl = jnp.broadcast_to(posl, (3 * NCHUNK, 8, 128))

    nbr = _knn(posl, posp)

    posx, posy, posz = posp[:, 0], posp[:, 1], posp[:, 2]
    s8, gidx = _geometry(posx, posy, posz, nbr)

    feats = jnp.concatenate([mass[:, None], vel, extra_feats], axis=-1)
    x = jnp.zeros((NP, 16), jnp.float32).at[:NPTS, :13].set(feats)

    def layer(x, Wc, Wd, bd, cin_pad, co_pad):
        w, bias = _prep_w(Wc, Wd, bd, cin_pad, co_pad)
        p, d = _mm(x, w, bias, 64 * co_pad, co_pad)
        p_rows = p.reshape(NP * 64, co_pad)
        c = _sc_fma(p_rows, gidx, s8, co_pad)
        return c, d

    c0, d0 = layer(x, Wc0, Wd0, bd0, 16, 32)
    x1 = jnp.concatenate([c0, d0], axis=1)
    c1, d1 = layer(x1, Wc1, Wd1, bd1, 64, 64)
    x2 = _combine(c1, d1, x1)
    c2, d2 = layer(x2, Wc2, Wd2, bd2, 64, 64)
    x3 = _combine(c2, d2, x2)
    c3, d3 = layer(x3, Wc3, Wd3, bd3, 64, 16)
    out = _combine(c3, d3)
    return out[:NPTS, :3]


# speedup vs baseline: 2.5897x; 1.0876x over previous
import functools

import jax
import jax.numpy as jnp
from jax import lax
from jax.experimental import pallas as pl
from jax.experimental.pallas import tpu as pltpu
from jax.experimental.pallas import tpu_sc as plsc

NPTS = 10000
NP = 10240
K_NB = 16
RADIUS = 1.5
KS = 4
NCHUNK = NP // 128
INTMAX = 2**31 - 1
D2CLIP = 2.26
KEYSCALE = float((2**18 - 1) / D2CLIP)
NW = 32
QPW = NP // NW
_SC_PARAMS = pltpu.CompilerParams(
    needs_layout_passes=False, use_tc_tiling_on_sc=False)


NLVL = 4
NSETS = 3


QG = 8
MLVL = 6


def _knn_phase1(posl_ref, qx, qy, qz, qidx, lane):
    pools = [[jnp.full((8, 128), INTMAX, jnp.int32) for _ in range(NLVL)]
             for _ in range(NSETS)]
    for v in range(NCHUNK):
        px = posl_ref[v]
        py = posl_ref[NCHUNK + v]
        pz = posl_ref[2 * NCHUNK + v]
        dx = qx - px
        dy = qy - py
        dz = qz - pz
        d2 = dx * dx + dy * dy + dz * dz
        j = v * 128 + lane
        q18 = (jnp.minimum(d2, D2CLIP) * KEYSCALE).astype(jnp.int32)
        key = jnp.left_shift(q18 - 2**17, 14) | j
        t = jnp.where(j == qidx, INTMAX, key)
        pool = pools[v % NSETS]
        for l in range(NLVL):
            lo = jnp.minimum(pool[l], t)
            t = jnp.maximum(pool[l], t)
            pool[l] = lo
    merged = pools[0] + [jnp.full((8, 128), INTMAX, jnp.int32)
                         for _ in range(MLVL - NLVL)]
    for s in range(1, NSETS):
        for l2 in range(NLVL):
            t = pools[s][l2]
            for l in range(MLVL):
                lo = jnp.minimum(merged[l], t)
                t = jnp.maximum(merged[l], t)
                merged[l] = lo
    return merged


def _knn_body(posl_ref, posq_ref, out_ref):
    i = pl.program_id(0)
    lane = lax.broadcasted_iota(jnp.int32, (8, 128), 1)
    gpools = []
    for g in range(QG):
        qx = posq_ref[g * 8:(g + 1) * 8, 0:1]
        qy = posq_ref[g * 8:(g + 1) * 8, 1:2]
        qz = posq_ref[g * 8:(g + 1) * 8, 2:3]
        qidx = (i * QG + g) * 8 + lax.broadcasted_iota(jnp.int32, (8, 1), 0)
        gpools.append(_knn_phase1(posl_ref, qx, qy, qz, qidx, lane))

    idxs = [[] for _ in range(QG)]
    for _ in range(K_NB):
        for g in range(QG):
            pool = gpools[g]
            m = jnp.min(pool[0], axis=1, keepdims=True)
            idxs[g].append(jnp.bitwise_and(m, 16383))
            c = pool[0] == m
            for l in range(MLVL - 1):
                pool[l] = jnp.where(c, pool[l + 1], pool[l])
            pool[MLVL - 1] = jnp.where(c, INTMAX, pool[MLVL - 1])
    out = jnp.concatenate(
        [jnp.concatenate(idxs[g], axis=1) for g in range(QG)], axis=0)
    out_ref[...] = out


def _knn(posl, posq):
    return pl.pallas_call(
        _knn_body,
        grid=(NP // (8 * QG),),
        in_specs=[
            pl.BlockSpec((3 * NCHUNK, 8, 128), lambda i: (0, 0, 0)),
            pl.BlockSpec((8 * QG, 3), lambda i: (i, 0)),
        ],
        out_specs=pl.BlockSpec((8 * QG, K_NB), lambda i: (i, 0)),
        out_shape=jax.ShapeDtypeStruct((NP, K_NB), jnp.int32),
    )(posl, posq)


def _sqrt_sc(a):
    b = lax.bitcast_convert_type(a, jnp.int32)
    y = lax.bitcast_convert_type(
        jnp.right_shift(b, 1) + jnp.int32(0x1FBD1DF5), jnp.float32)
    for _ in range(3):
        y = 0.5 * (y + a / y)
    return y


def _geom_body(posx_h, posy_h, posz_h, nbr_h, s8_h, gidx_h,
               px_v, py_v, pz_v, nbr_v, s8_v, gi_v):
    wid = lax.axis_index("s") * 2 + lax.axis_index("c")
    base = wid * QPW
    pltpu.sync_copy(posx_h, px_v)
    pltpu.sync_copy(posy_h, py_v)
    pltpu.sync_copy(posz_h, pz_v)
    pltpu.sync_copy(nbr_h.at[pl.ds(base * K_NB, QPW * K_NB)], nbr_v)

    cq = 16

    def chunk(ci, _):
        def one_q(iq, _):
            q = ci * cq + iq
            qsplat = jnp.full((16,), base + q, jnp.int32)
            qx = plsc.load_gather(px_v, [qsplat])
            qy = plsc.load_gather(py_v, [qsplat])
            qz = plsc.load_gather(pz_v, [qsplat])
            jdx = nbr_v[pl.ds(q * K_NB, K_NB)]
            pjx = plsc.load_gather(px_v, [jdx])
            pjy = plsc.load_gather(py_v, [jdx])
            pjz = plsc.load_gather(pz_v, [jdx])
            rx = (pjx - qx) * (1.0 / RADIUS)
            ry = (pjy - qy) * (1.0 / RADIUS)
            rz = (pjz - qz) * (1.0 / RADIUS)
            r2 = rx * rx + ry * ry + rz * rz
            r2c = jnp.minimum(jnp.maximum(r2, 0.0), 1.0)
            om = 1.0 - r2c
            win = om * om * om
            win = jnp.where(r2 <= 1.0, win, 0.0)
            norm2 = _sqrt_sc(jnp.maximum(r2, 1e-12))
            ninf = jnp.maximum(
                jnp.maximum(jnp.abs(rx), jnp.abs(ry)),
                jnp.maximum(jnp.abs(rz), 1e-12))
            s = norm2 / ninf
            ux = jnp.clip((jnp.clip(rx * s, -1.0, 1.0) + 1.0) * 1.5, 0.0, 3.0)
            uy = jnp.clip((jnp.clip(ry * s, -1.0, 1.0) + 1.0) * 1.5, 0.0, 3.0)
            uz = jnp.clip((jnp.clip(rz * s, -1.0, 1.0) + 1.0) * 1.5, 0.0, 3.0)
            ix = jnp.minimum(ux.astype(jnp.int32), 2)
            iy = jnp.minimum(uy.astype(jnp.int32), 2)
            iz = jnp.minimum(uz.astype(jnp.int32), 2)
            fx = ux - ix.astype(jnp.float32)
            fy = uy - iy.astype(jnp.float32)
            fz = uz - iz.astype(jnp.float32)
            j64 = jdx * 64
            gb = ix * 16 + iy * 4 + iz
            for c in range(8):
                cx, cy, cz = (c >> 2) & 1, (c >> 1) & 1, c & 1
                wx = fx if cx else 1.0 - fx
                wy = fy if cy else 1.0 - fy
                wz = fz if cz else 1.0 - fz
                w = win * wx * wy * wz
                g = gb + (cx * 16 + cy * 4 + cz)
                off = iq * 128 + c * 16
                s8_v[pl.ds(off, 16)] = w
                gi_v[pl.ds(off, 16)] = j64 + g
            return 0

        lax.fori_loop(0, cq, one_q, 0)
        hb = (base + ci * cq) * 128
        pltpu.sync_copy(s8_v, s8_h.at[pl.ds(hb, cq * 128)])
        pltpu.sync_copy(gi_v, gidx_h.at[pl.ds(hb, cq * 128)])
        return 0

    lax.fori_loop(0, QPW // cq, chunk, 0)


def _geometry(posx, posy, posz, nbr):
    mesh = plsc.VectorSubcoreMesh(core_axis_name="c", subcore_axis_name="s")
    f = pl.kernel(
        _geom_body,
        out_type=(
            jax.ShapeDtypeStruct((NP * 128,), jnp.float32),
            jax.ShapeDtypeStruct((NP * 128,), jnp.int32),
        ),
        mesh=mesh,
        compiler_params=_SC_PARAMS,
        scratch_types=[
            pltpu.VMEM((NP,), jnp.float32),
            pltpu.VMEM((NP,), jnp.float32),
            pltpu.VMEM((NP,), jnp.float32),
            pltpu.VMEM((QPW * K_NB,), jnp.int32),
            pltpu.VMEM((16 * 128,), jnp.float32),
            pltpu.VMEM((16 * 128,), jnp.int32),
        ],
    )
    return f(posx, posy, posz, nbr.reshape(NP * K_NB))


def _make_fma_body(co, cqs):
    nacc = co // 16
    nch = QPW // cqs

    def body(p_h, gidx_h, s8_h, out_h,
             idx0, idx1, s0, s1, rows0, rows1, acc_v, sem0, sem1):
        wid = lax.axis_index("s") * 2 + lax.axis_index("c")
        base = wid * QPW
        lane16 = lax.broadcasted_iota(jnp.int32, (16,), 0)

        def issue(ci, idx_v, s_v, rows_v, sem):
            q0 = base + ci * cqs
            pltpu.sync_copy(gidx_h.at[pl.ds(q0 * 128, cqs * 128)], idx_v)
            pltpu.sync_copy(s8_h.at[pl.ds(q0 * 128, cqs * 128)], s_v)
            for j in range(cqs):
                pltpu.async_copy(
                    p_h.at[idx_v.at[pl.ds(j * 128, 128)]],
                    rows_v.at[pl.ds(j * 128, 128)], sem)

        def drain(rows_v, sem):
            pltpu.make_async_copy(
                p_h.at[pl.ds(0, cqs * 128)], rows_v, sem).wait()

        def compute(ci, s_v, rows_v):
            q0 = base + ci * cqs

            def one_q(j, _):
                def grp(g, accs):
                    sbase = j * 128 + g * 16
                    for t in range(16):
                        sb = plsc.load_gather(
                            s_v, [jnp.full((16,), sbase + t, jnp.int32)])
                        accs = tuple(
                            accs[u] + sb * rows_v[sbase + t, pl.ds(u * 16, 16)]
                            for u in range(nacc))
                    return accs

                accs = lax.fori_loop(
                    0, 8, grp,
                    tuple(jnp.zeros((16,), jnp.float32) for _ in range(nacc)))
                for u in range(nacc):
                    acc_v[pl.ds(j * co + u * 16, 16)] = accs[u]
                return 0

            lax.fori_loop(0, cqs, one_q, 0)
            pltpu.sync_copy(acc_v, out_h.at[pl.ds(q0 * co, cqs * co)])

        issue(0, idx0, s0, rows0, sem0)

        def pair(cp, _):
            c0 = 2 * cp
            issue(c0 + 1, idx1, s1, rows1, sem1)
            drain(rows0, sem0)
            compute(c0, s0, rows0)

            @pl.when(c0 + 2 < nch)
            def _():
                issue(c0 + 2, idx0, s0, rows0, sem0)

            drain(rows1, sem1)
            compute(c0 + 1, s1, rows1)
            return 0

        lax.fori_loop(0, nch // 2, pair, 0)

    return body


def _sc_fma(p_rows, gidx, s8, co):
    cqs = 256 // co
    body = _make_fma_body(co, cqs)
    mesh = plsc.VectorSubcoreMesh(core_axis_name="c", subcore_axis_name="s")
    f = pl.kernel(
        body,
        out_type=jax.ShapeDtypeStruct((NP * co,), jnp.float32),
        mesh=mesh,
        compiler_params=_SC_PARAMS,
        scratch_types=[
            pltpu.VMEM((cqs * 128,), jnp.int32),
            pltpu.VMEM((cqs * 128,), jnp.int32),
            pltpu.VMEM((cqs * 128,), jnp.float32),
            pltpu.VMEM((cqs * 128,), jnp.float32),
            pltpu.VMEM((cqs * 128, co), jnp.float32),
            pltpu.VMEM((cqs * 128, co), jnp.float32),
            pltpu.VMEM((cqs * co,), jnp.float32),
            pltpu.SemaphoreType.DMA,
            pltpu.SemaphoreType.DMA,
        ],
    )
    return f(p_rows, gidx, s8).reshape(NP, co)


def _mm_body(x_ref, w_ref, b_ref, o1_ref, o2_ref, *, n1):
    res = jnp.dot(x_ref[...], w_ref[...], preferred_element_type=jnp.float32)
    o1_ref[...] = res[:, :n1]
    o2_ref[...] = res[:, n1:] + b_ref[...]


def _mm(x, w, bias2, n1, n2, bm=256):
    m, k = x.shape
    return pl.pallas_call(
        functools.partial(_mm_body, n1=n1),
        grid=(m // bm,),
        in_specs=[
            pl.BlockSpec((bm, k), lambda i: (i, 0)),
            pl.BlockSpec((k, n1 + n2), lambda i: (0, 0)),
            pl.BlockSpec((1, n2), lambda i: (0, 0)),
        ],
        out_specs=[
            pl.BlockSpec((bm, n1), lambda i: (i, 0)),
            pl.BlockSpec((bm, n2), lambda i: (i, 0)),
        ],
        out_shape=[
            jax.ShapeDtypeStruct((m, n1), jnp.float32),
            jax.ShapeDtypeStruct((m, n2), jnp.float32),
        ],
    )(x, w, bias2)


def _combine_body(c_ref, d_ref, r_ref, o_ref):
    o_ref[...] = jnp.maximum(c_ref[...] + d_ref[...] + r_ref[...], 0.0)


def _combine_body2(c_ref, d_ref, o_ref):
    o_ref[...] = jnp.maximum(c_ref[...] + d_ref[...], 0.0)


def _combine(c, d, res=None, bm=512):
    m, n = c.shape
    if res is None:
        return pl.pallas_call(
            _combine_body2,
            grid=(m // bm,),
            in_specs=[pl.BlockSpec((bm, n), lambda i: (i, 0))] * 2,
            out_specs=pl.BlockSpec((bm, n), lambda i: (i, 0)),
            out_shape=jax.ShapeDtypeStruct((m, n), jnp.float32),
        )(c, d)
    return pl.pallas_call(
        _combine_body,
        grid=(m // bm,),
        in_specs=[pl.BlockSpec((bm, n), lambda i: (i, 0))] * 3,
        out_specs=pl.BlockSpec((bm, n), lambda i: (i, 0)),
        out_shape=jax.ShapeDtypeStruct((m, n), jnp.float32),
    )(c, d, res)


def _prep_w(Wc, Wd, bd, cin_pad, co_pad):
    ks3, cin, co = Wc.shape
    wc = jnp.zeros((ks3, cin_pad, co_pad), jnp.float32).at[:, :cin, :co].set(Wc)
    wcf = wc.transpose(1, 0, 2).reshape(cin_pad, ks3 * co_pad)
    wd = jnp.zeros((cin_pad, co_pad), jnp.float32).at[:cin, :co].set(Wd)
    w = jnp.concatenate([wcf, wd], axis=1)
    bias = jnp.zeros((1, co_pad), jnp.float32).at[0, :co].set(bd)
    return w, bias


def kernel(pos, vel, mass, extra_feats,
           Wc0, Wd0, bd0, Wc1, Wd1, bd1, Wc2, Wd2, bd2, Wc3, Wd3, bd3):
    posp = jnp.full((NP, 3), 1e4, jnp.float32).at[:NPTS].set(pos)
    posl = posp.T.reshape(3, NCHUNK, 1, 128).reshape(3 * NCHUNK, 1, 128)
    posl = jnp.broadcast_to(posl, (3 * NCHUNK, 8, 128))

    nbr = _knn(posl, posp)

    posx, posy, posz = posp[:, 0], posp[:, 1], posp[:, 2]
    s8, gidx = _geometry(posx, posy, posz, nbr)

    feats = jnp.concatenate([mass[:, None], vel, extra_feats], axis=-1)
    x = jnp.zeros((NP, 16), jnp.float32).at[:NPTS, :13].set(feats)

    def layer(x, Wc, Wd, bd, cin_pad, co_pad):
        w, bias = _prep_w(Wc, Wd, bd, cin_pad, co_pad)
        p, d = _mm(x, w, bias, 64 * co_pad, co_pad)
        p_rows = p.reshape(NP * 64, co_pad)
        c = _sc_fma(p_rows, gidx, s8, co_pad)
        return c, d

    c0, d0 = layer(x, Wc0, Wd0, bd0, 16, 32)
    x1 = jnp.concatenate([c0, d0], axis=1)
    c1, d1 = layer(x1, Wc1, Wd1, bd1, 64, 64)
    x2 = _combine(c1, d1, x1)
    c2, d2 = layer(x2, Wc2, Wd2, bd2, 64, 64)
    x3 = _combine(c2, d2, x2)
    c3, d3 = layer(x3, Wc3, Wd3, bd3, 64, 16)
    out = _combine(c3, d3)
    return out[:NPTS, :3]
